# Initial kernel scaffold; baseline (speedup 1.0000x reference)
import jax
import jax.numpy as jnp
from jax.experimental import pallas as pl


def kernel(x, W_enc, b_enc, W_dec, b_dec):
    raise NotImplementedError("write your pallas kernel here")


# speedup vs baseline: 3.2455x; 3.2455x over previous
import functools

import jax
import jax.numpy as jnp
from jax.experimental import pallas as pl
from jax.experimental.pallas import tpu as pltpu

_POS_INF_BITS = 0x7F800000


def _sae_kernel_body(C, BR, BC, K,
                     x_ref, we_ref, be_ref, wd_ref, bd_ref,
                     zpre_ref, zsp_ref, xrec_ref,
                     zscr_ref, acc_ref, thr_ref):
    c = pl.program_id(1)

    @pl.when(c < C)
    def _encode():
        zp = jax.lax.dot_general(
            x_ref[...], we_ref[...], (((1,), (0,)), ((), ())),
            precision=jax.lax.Precision.DEFAULT,
            preferred_element_type=jnp.float32,
        ) + be_ref[...]
        zpre_ref[...] = zp
        zscr_ref[c] = jnp.maximum(zp, 0.0)

    @pl.when(c == C - 1)
    def _threshold():
        def body(_, carry):
            lo, hi = carry
            mid = lo + (hi - lo) // 2
            midf = jax.lax.bitcast_convert_type(mid, jnp.float32)

            def cbody(i, acc):
                zc = zscr_ref[i]
                return acc + jnp.sum((zc >= midf).astype(jnp.int32),
                                     axis=1)[:, None]

            cnt = jax.lax.fori_loop(0, C, cbody,
                                    jnp.zeros((BR, 1), jnp.int32))
            ge = cnt >= K
            lo = jnp.where(ge, mid, lo)
            hi = jnp.where(ge, hi, mid)
            return lo, hi

        lo0 = jnp.zeros((BR, 1), jnp.int32)
        hi0 = jnp.full((BR, 1), _POS_INF_BITS, jnp.int32)
        lo, _ = jax.lax.fori_loop(0, 31, body, (lo0, hi0))
        thr_ref[...] = jax.lax.bitcast_convert_type(lo, jnp.float32)

    @pl.when(c >= C)
    def _decode():
        j = c - C
        zc = zscr_ref[j]
        zs = jnp.where(zc >= thr_ref[...], zc, 0.0)
        zsp_ref[...] = zs
        part = jax.lax.dot_general(
            zs, wd_ref[...], (((1,), (0,)), ((), ())),
            precision=jax.lax.Precision.HIGHEST,
            preferred_element_type=jnp.float32,
        )

        @pl.when(j == 0)
        def _():
            acc_ref[...] = part + bd_ref[...]

        @pl.when(j > 0)
        def _():
            acc_ref[...] = acc_ref[...] + part

        @pl.when(j == C - 1)
        def _():
            xrec_ref[...] = acc_ref[...]


def _build_call(N, D, S, K, BR, BC, interpret=False):
    C = S // BC
    R = N // BR
    body = functools.partial(_sae_kernel_body, C, BR, BC, K)
    grid = (R, 2 * C)

    in_specs = [
        pl.BlockSpec((BR, D), lambda r, c: (r, 0)),
        pl.BlockSpec((D, BC), lambda r, c: (0, jnp.minimum(c, C - 1))),
        pl.BlockSpec((1, BC), lambda r, c: (0, jnp.minimum(c, C - 1))),
        pl.BlockSpec((BC, D), lambda r, c: (jnp.maximum(c - C, 0), 0)),
        pl.BlockSpec((1, D), lambda r, c: (0, 0)),
    ]
    out_specs = [
        pl.BlockSpec((BR, BC), lambda r, c: (r, jnp.minimum(c, C - 1))),
        pl.BlockSpec((BR, BC), lambda r, c: (r, jnp.maximum(c - C, 0))),
        pl.BlockSpec((BR, D), lambda r, c: (r, 0)),
    ]
    out_shape = [
        jax.ShapeDtypeStruct((N, S), jnp.float32),
        jax.ShapeDtypeStruct((N, S), jnp.float32),
        jax.ShapeDtypeStruct((N, D), jnp.float32),
    ]
    scratch_shapes = [
        pltpu.VMEM((C, BR, BC), jnp.float32),
        pltpu.VMEM((BR, D), jnp.float32),
        pltpu.VMEM((BR, 1), jnp.float32),
    ]
    return pl.pallas_call(
        body,
        grid=grid,
        in_specs=in_specs,
        out_specs=out_specs,
        out_shape=out_shape,
        scratch_shapes=scratch_shapes,
        compiler_params=pltpu.CompilerParams(
            dimension_semantics=("arbitrary", "arbitrary"),
        ),
        interpret=interpret,
    )


def _pick_block(n, target):
    b = min(n, target)
    while n % b:
        b -= 1
    return b


def kernel(x, W_enc, b_enc, W_dec, b_dec, *, _interpret=False):
    N, D = x.shape
    S = W_enc.shape[1]
    K = 32
    BR = _pick_block(N, 512)
    BC = _pick_block(S, 512)
    call = _build_call(N, D, S, K, BR, BC, interpret=_interpret)
    z_pre, z_sparse, x_recon = call(
        x, W_enc, b_enc.reshape(1, S), W_dec, b_dec.reshape(1, D))
    return (x_recon, z_sparse, z_pre)


# speedup vs baseline: 3.8162x; 1.1758x over previous
import functools

import jax
import jax.numpy as jnp
from jax.experimental import pallas as pl
from jax.experimental.pallas import tpu as pltpu

_POS_INF_BITS = 0x7F800000


def _sae_kernel_body(C, BR, BC, K,
                     x_ref, we_ref, be_ref, wd_ref, bd_ref,
                     zpre_ref, zsp_ref, xrec_ref,
                     zscr_ref, acc_ref, thr_ref):
    c = pl.program_id(1)

    @pl.when(c < C)
    def _encode():
        zp = jax.lax.dot_general(
            x_ref[...], we_ref[...], (((1,), (0,)), ((), ())),
            precision=jax.lax.Precision.DEFAULT,
            preferred_element_type=jnp.float32,
        ) + be_ref[...]
        zpre_ref[...] = zp
        zscr_ref[c] = jnp.maximum(zp, 0.0)

    @pl.when(c == C - 1)
    def _threshold():
        def body(_, carry):
            lo, hi = carry
            mid = lo + (hi - lo) // 2
            midf = jax.lax.bitcast_convert_type(mid, jnp.float32)

            def cbody(i, acc):
                zc = zscr_ref[i]
                return acc + jnp.sum((zc >= midf).astype(jnp.int32),
                                     axis=1)[:, None]

            cnt = jax.lax.fori_loop(0, C, cbody,
                                    jnp.zeros((BR, 1), jnp.int32))
            ge = cnt >= K
            lo = jnp.where(ge, mid, lo)
            hi = jnp.where(ge, hi, mid)
            return lo, hi

        lo0 = jnp.zeros((BR, 1), jnp.int32)
        hi0 = jnp.full((BR, 1), _POS_INF_BITS, jnp.int32)
        lo, _ = jax.lax.fori_loop(0, 31, body, (lo0, hi0))
        thr_ref[...] = jax.lax.bitcast_convert_type(lo, jnp.float32)

    @pl.when(c >= C)
    def _decode():
        j = c - C
        zc = zscr_ref[j]
        zs = jnp.where(zc >= thr_ref[...], zc, 0.0)
        zsp_ref[...] = zs
        part = jax.lax.dot_general(
            zs.astype(jnp.bfloat16), wd_ref[...], (((1,), (0,)), ((), ())),
            precision=jax.lax.Precision.DEFAULT,
            preferred_element_type=jnp.float32,
        )

        @pl.when(j == 0)
        def _():
            acc_ref[...] = part + bd_ref[...]

        @pl.when(j > 0)
        def _():
            acc_ref[...] = acc_ref[...] + part

        @pl.when(j == C - 1)
        def _():
            xrec_ref[...] = acc_ref[...]


def _build_call(N, D, S, K, BR, BC, interpret=False):
    C = S // BC
    R = N // BR
    body = functools.partial(_sae_kernel_body, C, BR, BC, K)
    grid = (R, 2 * C)

    in_specs = [
        pl.BlockSpec((BR, D), lambda r, c: (r, 0)),
        pl.BlockSpec((D, BC), lambda r, c: (0, jnp.minimum(c, C - 1))),
        pl.BlockSpec((1, BC), lambda r, c: (0, jnp.minimum(c, C - 1))),
        pl.BlockSpec((BC, D), lambda r, c: (jnp.maximum(c - C, 0), 0)),
        pl.BlockSpec((1, D), lambda r, c: (0, 0)),
    ]
    out_specs = [
        pl.BlockSpec((BR, BC), lambda r, c: (r, jnp.minimum(c, C - 1))),
        pl.BlockSpec((BR, BC), lambda r, c: (r, jnp.maximum(c - C, 0))),
        pl.BlockSpec((BR, D), lambda r, c: (r, 0)),
    ]
    out_shape = [
        jax.ShapeDtypeStruct((N, S), jnp.float32),
        jax.ShapeDtypeStruct((N, S), jnp.float32),
        jax.ShapeDtypeStruct((N, D), jnp.float32),
    ]
    scratch_shapes = [
        pltpu.VMEM((C, BR, BC), jnp.float32),
        pltpu.VMEM((BR, D), jnp.float32),
        pltpu.VMEM((BR, 1), jnp.float32),
    ]
    return pl.pallas_call(
        body,
        grid=grid,
        in_specs=in_specs,
        out_specs=out_specs,
        out_shape=out_shape,
        scratch_shapes=scratch_shapes,
        compiler_params=pltpu.CompilerParams(
            dimension_semantics=("arbitrary", "arbitrary"),
        ),
        interpret=interpret,
    )


def _pick_block(n, target):
    b = min(n, target)
    while n % b:
        b -= 1
    return b


def kernel(x, W_enc, b_enc, W_dec, b_dec, *, _interpret=False):
    N, D = x.shape
    S = W_enc.shape[1]
    K = 32
    BR = _pick_block(N, 512)
    BC = _pick_block(S, 512)
    call = _build_call(N, D, S, K, BR, BC, interpret=_interpret)
    z_pre, z_sparse, x_recon = call(
        x.astype(jnp.bfloat16), W_enc.astype(jnp.bfloat16),
        b_enc.reshape(1, S), W_dec.astype(jnp.bfloat16), b_dec.reshape(1, D))
    return (x_recon, z_sparse, z_pre)


# speedup vs baseline: 4.9416x; 1.2949x over previous
import functools

import jax
import jax.numpy as jnp
from jax.experimental import pallas as pl
from jax.experimental.pallas import tpu as pltpu

_POS_INF_BITS = 0x7F800000


def _sae_kernel_body(C, BR, BC, K,
                     x_ref, we_ref, be_ref, wd_ref, bd_ref,
                     zpre_ref, zsp_ref, xrec_ref,
                     zscr_ref, acc_ref, thr_ref, m_ref):
    c = pl.program_id(1)

    @pl.when(c < C)
    def _encode():
        zp = jax.lax.dot_general(
            x_ref[...], we_ref[...], (((1,), (0,)), ((), ())),
            precision=jax.lax.Precision.DEFAULT,
            preferred_element_type=jnp.float32,
        ) + be_ref[...]
        zpre_ref[...] = zp
        zr = jnp.maximum(zp, 0.0)
        zscr_ref[c] = zr
        m_ref[c] = jnp.max(zr, axis=1, keepdims=True)

    @pl.when(c == C - 1)
    def _threshold():
        m = m_ref[...]
        lo0 = jax.lax.bitcast_convert_type(jnp.min(m, axis=0), jnp.int32)
        hi0 = jax.lax.bitcast_convert_type(jnp.max(m, axis=0), jnp.int32) + 1
        def cond(state):
            i, lo, hi = state
            return jnp.logical_and(i < 31,
                                   jnp.logical_not(jnp.all(hi - lo <= 1)))

        def body(state):
            i, lo, hi = state
            mid = lo + (hi - lo) // 2
            midf = jax.lax.bitcast_convert_type(mid, jnp.float32)

            def cbody(j, acc):
                zc = zscr_ref[j]
                return acc + jnp.sum((zc >= midf).astype(jnp.int32),
                                     axis=1)[:, None]

            cnt = jax.lax.fori_loop(0, C, cbody,
                                    jnp.zeros((BR, 1), jnp.int32))
            ge = cnt >= K
            lo = jnp.where(ge, mid, lo)
            hi = jnp.where(cnt == K, mid, jnp.where(ge, hi, mid))
            return i + 1, lo, hi

        _, lo, _ = jax.lax.while_loop(cond, body, (0, lo0, hi0))
        thr_ref[...] = jax.lax.bitcast_convert_type(lo, jnp.float32)

    @pl.when(c >= C)
    def _decode():
        j = c - C
        zc = zscr_ref[j]
        zs = jnp.where(zc >= thr_ref[...], zc, 0.0)
        zsp_ref[...] = zs
        part = jax.lax.dot_general(
            zs.astype(jnp.bfloat16), wd_ref[...], (((1,), (0,)), ((), ())),
            precision=jax.lax.Precision.DEFAULT,
            preferred_element_type=jnp.float32,
        )

        @pl.when(j == 0)
        def _():
            acc_ref[...] = part + bd_ref[...]

        @pl.when(j > 0)
        def _():
            acc_ref[...] = acc_ref[...] + part

        @pl.when(j == C - 1)
        def _():
            xrec_ref[...] = acc_ref[...]


def _build_call(N, D, S, K, BR, BC, interpret=False):
    C = S // BC
    R = N // BR
    body = functools.partial(_sae_kernel_body, C, BR, BC, K)
    grid = (R, 2 * C)

    in_specs = [
        pl.BlockSpec((BR, D), lambda r, c: (r, 0)),
        pl.BlockSpec((D, BC), lambda r, c: (0, jnp.minimum(c, C - 1))),
        pl.BlockSpec((1, BC), lambda r, c: (0, jnp.minimum(c, C - 1))),
        pl.BlockSpec((BC, D), lambda r, c: (jnp.maximum(c - C, 0), 0)),
        pl.BlockSpec((1, D), lambda r, c: (0, 0)),
    ]
    out_specs = [
        pl.BlockSpec((BR, BC), lambda r, c: (r, jnp.minimum(c, C - 1))),
        pl.BlockSpec((BR, BC), lambda r, c: (r, jnp.maximum(c - C, 0))),
        pl.BlockSpec((BR, D), lambda r, c: (r, 0)),
    ]
    out_shape = [
        jax.ShapeDtypeStruct((N, S), jnp.float32),
        jax.ShapeDtypeStruct((N, S), jnp.float32),
        jax.ShapeDtypeStruct((N, D), jnp.float32),
    ]
    assert C >= K, "chunk-max lower bound needs at least K chunks"
    scratch_shapes = [
        pltpu.VMEM((C, BR, BC), jnp.float32),
        pltpu.VMEM((BR, D), jnp.float32),
        pltpu.VMEM((BR, 1), jnp.float32),
        pltpu.VMEM((C, BR, 1), jnp.float32),
    ]
    return pl.pallas_call(
        body,
        grid=grid,
        in_specs=in_specs,
        out_specs=out_specs,
        out_shape=out_shape,
        scratch_shapes=scratch_shapes,
        compiler_params=pltpu.CompilerParams(
            dimension_semantics=("arbitrary", "arbitrary"),
        ),
        interpret=interpret,
    )


def _pick_block(n, target):
    b = min(n, target)
    while n % b:
        b -= 1
    return b


def kernel(x, W_enc, b_enc, W_dec, b_dec, *, _interpret=False):
    N, D = x.shape
    S = W_enc.shape[1]
    K = 32
    BR = _pick_block(N, 512)
    BC = _pick_block(S, 512)
    call = _build_call(N, D, S, K, BR, BC, interpret=_interpret)
    z_pre, z_sparse, x_recon = call(
        x.astype(jnp.bfloat16), W_enc.astype(jnp.bfloat16),
        b_enc.reshape(1, S), W_dec.astype(jnp.bfloat16), b_dec.reshape(1, D))
    return (x_recon, z_sparse, z_pre)


# speedup vs baseline: 9.1386x; 1.8493x over previous
import functools

import jax
import jax.numpy as jnp
from jax.experimental import pallas as pl
from jax.experimental.pallas import tpu as pltpu


def _sae_kernel_body(C, BR, BC, K, GPC,
                     x_ref, we_ref, be_ref, wd_ref, bd_ref,
                     zpre_ref, zsp_ref, xrec_ref,
                     zscr_ref, acc_ref, thr_ref, m_ref):
    c = pl.program_id(1)
    GW = BC // GPC

    @pl.when(c < C)
    def _encode():
        zp = jax.lax.dot_general(
            x_ref[...], we_ref[...], (((1,), (0,)), ((), ())),
            precision=jax.lax.Precision.DEFAULT,
            preferred_element_type=jnp.float32,
        ) + be_ref[...]
        zpre_ref[...] = zp
        zr = jnp.maximum(zp, 0.0)
        zscr_ref[c] = zr
        for g in range(GPC):
            m_ref[c * GPC + g] = jnp.max(zr[:, g * GW:(g + 1) * GW],
                                         axis=1, keepdims=True)

    @pl.when(c == C - 1)
    def _threshold():
        m = m_ref[...]
        lo0 = jax.lax.bitcast_convert_type(jnp.min(m, axis=0), jnp.int32)
        hi0 = jax.lax.bitcast_convert_type(jnp.max(m, axis=0), jnp.int32) + 1

        def cond(state):
            i, lo, hi = state
            return jnp.logical_and(i < 31,
                                   jnp.logical_not(jnp.all(hi - lo <= 1)))

        def body(state):
            i, lo, hi = state
            mid = lo + (hi - lo) // 2
            midf = jax.lax.bitcast_convert_type(mid, jnp.float32)
            cnt = jnp.zeros((BR, 1), jnp.int32)
            for j in range(C):
                zc = zscr_ref[j]
                cnt = cnt + jnp.sum((zc >= midf).astype(jnp.int32),
                                    axis=1)[:, None]
            ge = cnt >= K
            lo = jnp.where(ge, mid, lo)
            hi = jnp.where(cnt == K, mid, jnp.where(ge, hi, mid))
            return i + 1, lo, hi

        _, lo, _ = jax.lax.while_loop(cond, body, (0, lo0, hi0))
        thr_ref[...] = jax.lax.bitcast_convert_type(lo, jnp.float32)

    @pl.when(c >= C)
    def _decode():
        j = c - C
        zc = zscr_ref[j]
        zs = jnp.where(zc >= thr_ref[...], zc, 0.0)
        zsp_ref[...] = zs
        part = jax.lax.dot_general(
            zs.astype(jnp.bfloat16), wd_ref[...], (((1,), (0,)), ((), ())),
            precision=jax.lax.Precision.DEFAULT,
            preferred_element_type=jnp.float32,
        )

        @pl.when(j == 0)
        def _():
            acc_ref[...] = part + bd_ref[...]

        @pl.when(j > 0)
        def _():
            acc_ref[...] = acc_ref[...] + part

        @pl.when(j == C - 1)
        def _():
            xrec_ref[...] = acc_ref[...]


def _build_call(N, D, S, K, BR, BC, interpret=False):
    C = S // BC
    R = N // BR
    GPC = max(1, -(-K // C))
    assert C * GPC >= K and BC % GPC == 0
    body = functools.partial(_sae_kernel_body, C, BR, BC, K, GPC)
    grid = (R, 2 * C)

    in_specs = [
        pl.BlockSpec((BR, D), lambda r, c: (r, 0)),
        pl.BlockSpec((D, BC), lambda r, c: (0, jnp.minimum(c, C - 1))),
        pl.BlockSpec((1, BC), lambda r, c: (0, jnp.minimum(c, C - 1))),
        pl.BlockSpec((BC, D), lambda r, c: (jnp.maximum(c - C, 0), 0)),
        pl.BlockSpec((1, D), lambda r, c: (0, 0)),
    ]
    out_specs = [
        pl.BlockSpec((BR, BC), lambda r, c: (r, jnp.minimum(c, C - 1))),
        pl.BlockSpec((BR, BC), lambda r, c: (r, jnp.maximum(c - C, 0))),
        pl.BlockSpec((BR, D), lambda r, c: (r, 0)),
    ]
    out_shape = [
        jax.ShapeDtypeStruct((N, S), jnp.float32),
        jax.ShapeDtypeStruct((N, S), jnp.float32),
        jax.ShapeDtypeStruct((N, D), jnp.float32),
    ]
    scratch_shapes = [
        pltpu.VMEM((C, BR, BC), jnp.float32),
        pltpu.VMEM((BR, D), jnp.float32),
        pltpu.VMEM((BR, 1), jnp.float32),
        pltpu.VMEM((C * GPC, BR, 1), jnp.float32),
    ]
    return pl.pallas_call(
        body,
        grid=grid,
        in_specs=in_specs,
        out_specs=out_specs,
        out_shape=out_shape,
        scratch_shapes=scratch_shapes,
        compiler_params=pltpu.CompilerParams(
            dimension_semantics=("arbitrary", "arbitrary"),
            vmem_limit_bytes=100 * 1024 * 1024,
        ),
        interpret=interpret,
    )


def _pick_block(n, target):
    b = min(n, target)
    while n % b:
        b -= 1
    return b


def kernel(x, W_enc, b_enc, W_dec, b_dec, *, _interpret=False):
    N, D = x.shape
    S = W_enc.shape[1]
    K = 32
    BR = _pick_block(N, 512)
    BC = _pick_block(S, 1024)
    call = _build_call(N, D, S, K, BR, BC, interpret=_interpret)
    z_pre, z_sparse, x_recon = call(
        x.astype(jnp.bfloat16), W_enc.astype(jnp.bfloat16),
        b_enc.reshape(1, S), W_dec.astype(jnp.bfloat16), b_dec.reshape(1, D))
    return (x_recon, z_sparse, z_pre)


# speedup vs baseline: 10.2054x; 1.1167x over previous
import functools

import jax
import jax.numpy as jnp
from jax.experimental import pallas as pl
from jax.experimental.pallas import tpu as pltpu


def _sae_kernel_body(C, BR, BC, K, GPC,
                     x_ref, we_ref, be_ref, wd_ref, bd_ref,
                     zpre_ref, zsp_ref, xrec_ref,
                     zscr_ref, acc_ref, thr_ref, m_ref):
    c = pl.program_id(1)
    GW = BC // GPC

    @pl.when(c < C)
    def _encode():
        zp = jax.lax.dot_general(
            x_ref[...], we_ref[...],
            (((1,), (0,)), ((), ())),
            precision=jax.lax.Precision.DEFAULT,
            preferred_element_type=jnp.float32,
        ) + be_ref[...]
        zpre_ref[...] = zp
        zr = jnp.maximum(zp, 0.0)
        zscr_ref[c] = zr
        for g in range(GPC):
            m_ref[c * GPC + g] = jnp.max(zr[:, g * GW:(g + 1) * GW],
                                         axis=1, keepdims=True)

    @pl.when(c == C - 1)
    def _threshold():
        m = m_ref[...]
        lo0 = jax.lax.bitcast_convert_type(jnp.min(m, axis=0), jnp.int32)
        hi0 = jax.lax.bitcast_convert_type(jnp.max(m, axis=0), jnp.int32) + 1

        def cond(state):
            i, lo, hi = state
            return jnp.logical_and(i < 31,
                                   jnp.logical_not(jnp.all(hi - lo <= 1)))

        def body(state):
            i, lo, hi = state
            mid = lo + (hi - lo) // 2
            midf = jax.lax.bitcast_convert_type(mid, jnp.float32)
            parts = []
            for rb in range(BR // 128):
                rsl = slice(rb * 128, (rb + 1) * 128)
                mrb = midf[rsl]
                acc = jnp.zeros((128, 128), jnp.int32)
                for j in range(C):
                    zc = zscr_ref[j, rsl, :]
                    for k in range(BC // 128):
                        acc = acc + (zc[:, k * 128:(k + 1) * 128]
                                     >= mrb).astype(jnp.int32)
                parts.append(jnp.sum(acc, axis=1)[:, None])
            cnt = jnp.concatenate(parts, axis=0)
            ge = cnt >= K
            lo = jnp.where(ge, mid, lo)
            hi = jnp.where(cnt == K, mid, jnp.where(ge, hi, mid))
            return i + 1, lo, hi

        _, lo, _ = jax.lax.while_loop(cond, body, (0, lo0, hi0))
        thr_ref[...] = jax.lax.bitcast_convert_type(lo, jnp.float32)

    @pl.when(c >= C)
    def _decode():
        j = c - C
        zc = zscr_ref[j]
        zs = jnp.where(zc >= thr_ref[...], zc, 0.0)
        zsp_ref[...] = zs
        part = jax.lax.dot_general(
            zs.astype(jnp.bfloat16), wd_ref[...],
            (((1,), (0,)), ((), ())),
            precision=jax.lax.Precision.DEFAULT,
            preferred_element_type=jnp.float32,
        )

        @pl.when(j == 0)
        def _():
            acc_ref[...] = part + bd_ref[...]

        @pl.when(j > 0)
        def _():
            acc_ref[...] = acc_ref[...] + part

        @pl.when(j == C - 1)
        def _():
            xrec_ref[...] = acc_ref[...]


def _build_call(N, D, S, K, BR, BC, interpret=False):
    C = S // BC
    R = N // BR
    GPC = max(1, -(-K // C))
    assert C * GPC >= K and BC % GPC == 0
    body = functools.partial(_sae_kernel_body, C, BR, BC, K, GPC)
    grid = (R, 2 * C)

    in_specs = [
        pl.BlockSpec((BR, D), lambda r, c: (r, 0)),
        pl.BlockSpec((D, BC), lambda r, c: (0, jnp.minimum(c, C - 1))),
        pl.BlockSpec((1, BC), lambda r, c: (0, jnp.minimum(c, C - 1))),
        pl.BlockSpec((BC, D), lambda r, c: (jnp.maximum(c - C, 0), 0)),
        pl.BlockSpec((1, D), lambda r, c: (0, 0)),
    ]
    out_specs = [
        pl.BlockSpec((BR, BC), lambda r, c: (r, jnp.minimum(c, C - 1))),
        pl.BlockSpec((BR, BC), lambda r, c: (r, jnp.maximum(c - C, 0))),
        pl.BlockSpec((BR, D), lambda r, c: (r, 0)),
    ]
    out_shape = [
        jax.ShapeDtypeStruct((N, S), jnp.float32),
        jax.ShapeDtypeStruct((N, S), jnp.float32),
        jax.ShapeDtypeStruct((N, D), jnp.float32),
    ]
    scratch_shapes = [
        pltpu.VMEM((C, BR, BC), jnp.float32),
        pltpu.VMEM((BR, D), jnp.float32),
        pltpu.VMEM((BR, 1), jnp.float32),
        pltpu.VMEM((C * GPC, BR, 1), jnp.float32),
    ]
    return pl.pallas_call(
        body,
        grid=grid,
        in_specs=in_specs,
        out_specs=out_specs,
        out_shape=out_shape,
        scratch_shapes=scratch_shapes,
        compiler_params=pltpu.CompilerParams(
            dimension_semantics=("arbitrary", "arbitrary"),
            vmem_limit_bytes=112 * 1024 * 1024,
        ),
        interpret=interpret,
    )


def _pick_block(n, target):
    b = min(n, target)
    while n % b:
        b -= 1
    return b


def kernel(x, W_enc, b_enc, W_dec, b_dec, *, _interpret=False):
    N, D = x.shape
    S = W_enc.shape[1]
    K = 32
    BR = _pick_block(N, 512)
    BC = _pick_block(S, 1024)
    call = _build_call(N, D, S, K, BR, BC, interpret=_interpret)
    z_pre, z_sparse, x_recon = call(
        x.astype(jnp.bfloat16), W_enc.astype(jnp.bfloat16),
        b_enc.reshape(1, S), W_dec.astype(jnp.bfloat16), b_dec.reshape(1, D))
    return (x_recon, z_sparse, z_pre)
